# Initial kernel scaffold; baseline (speedup 1.0000x reference)
import jax
import jax.numpy as jnp
from jax.experimental import pallas as pl


def kernel(X, pair_indices, d_ij, radial_feature_vector, atomic_charges, W1, b1, W2, b2, W3, b3, Wt):
    raise NotImplementedError("write your pallas kernel here")


# speedup vs baseline: 29.0813x; 29.0813x over previous
import numpy as np
import jax
import jax.numpy as jnp
from jax.experimental import pallas as pl
from jax.experimental.pallas import tpu as pltpu

_F = 64
_D = 9 * _F
_CUT = 5.0


def _largest_divisor_leq(n, cap):
    for d in range(min(cap, n), 0, -1):
        if n % d == 0:
            return d
    return 1


def _node_prep_kernel(xc_ref, w0_ref, w1_ref, w2_ref, p_ref, xn_ref):
    x = xc_ref[...]
    norm = jnp.sum(x * x, axis=0)
    inv = 1.0 / (norm + 1.0)
    xn = x * inv[None]
    xn_ref[...] = xn
    dm = (xn[0] + xn[4] + xn[8]) * (1.0 / 3.0)
    a01 = 0.5 * (xn[1] - xn[3])
    a02 = 0.5 * (xn[2] - xn[6])
    a12 = 0.5 * (xn[5] - xn[7])
    s00 = xn[0] - dm
    s11 = xn[4] - dm
    s01 = 0.5 * (xn[1] + xn[3])
    s02 = 0.5 * (xn[2] + xn[6])
    s12 = 0.5 * (xn[5] + xn[7])
    w0 = w0_ref[...]
    w1 = w1_ref[...]
    w2 = w2_ref[...]

    def mm(v, w):
        return jnp.dot(v, w, preferred_element_type=jnp.float32)

    p_ref[:, 0 * _F:1 * _F] = mm(dm, w0)
    p_ref[:, 1 * _F:2 * _F] = mm(a01, w1)
    p_ref[:, 2 * _F:3 * _F] = mm(a02, w1)
    p_ref[:, 3 * _F:4 * _F] = mm(a12, w1)
    p_ref[:, 4 * _F:5 * _F] = mm(s00, w2)
    p_ref[:, 5 * _F:6 * _F] = mm(s01, w2)
    p_ref[:, 6 * _F:7 * _F] = mm(s02, w2)
    p_ref[:, 7 * _F:8 * _F] = mm(s11, w2)
    p_ref[:, 8 * _F:9 * _F] = mm(s12, w2)


def _msg_kernel(rbf_ref, d_ref, sc_ref, gt_ref, p_ref,
                w1_ref, b1_ref, w2_ref, b2_ref, w3_ref, b3_ref, mc_ref,
                scale_ref):
    @pl.when(pl.program_id(0) == 0)
    def _init():
        mc_ref[...] = jnp.zeros_like(mc_ref)

    h = rbf_ref[...]
    h = jax.nn.silu(jnp.dot(h, w1_ref[...], preferred_element_type=jnp.float32)
                    + b1_ref[...])
    h = jax.nn.silu(jnp.dot(h, w2_ref[...], preferred_element_type=jnp.float32)
                    + b2_ref[...])
    h = jax.nn.silu(jnp.dot(h, w3_ref[...], preferred_element_type=jnp.float32)
                    + b3_ref[...])
    d = d_ref[...]
    c = 0.5 * (jnp.cos(d * (np.pi / _CUT)) + 1.0)
    c = c * (d < _CUT).astype(jnp.float32)
    r = h * c
    r0 = r[:, 0 * _F:1 * _F]
    r1 = r[:, 1 * _F:2 * _F]
    r2 = r[:, 2 * _F:3 * _F]
    scale_ref[...] = jnp.concatenate([r0, r1, r1, r1, r2, r2, r2, r2, r2],
                                     axis=1)
    eb = r.shape[0]

    def body(i, carry):
        g = gt_ref[0, 0, i]
        s = sc_ref[0, 0, i]
        row = p_ref[pl.ds(g, 1), :]
        m = row * scale_ref[pl.ds(i, 1), :]
        mc_ref[pl.ds(s, 1), :] = mc_ref[pl.ds(s, 1), :] + m
        return carry

    jax.lax.fori_loop(0, eb, body, 0)


def _planes_from_packed(p):
    i = p[:, 0 * _F:1 * _F]
    a01 = p[:, 1 * _F:2 * _F]
    a02 = p[:, 2 * _F:3 * _F]
    a12 = p[:, 3 * _F:4 * _F]
    s00 = p[:, 4 * _F:5 * _F]
    s01 = p[:, 5 * _F:6 * _F]
    s02 = p[:, 6 * _F:7 * _F]
    s11 = p[:, 7 * _F:8 * _F]
    s12 = p[:, 8 * _F:9 * _F]
    s22 = -(s00 + s11)
    return [i + s00, a01 + s01, a02 + s02,
            -a01 + s01, i + s11, a12 + s12,
            -a02 + s02, -a12 + s12, i + s22]


def _matmul3(a, b):
    out = []
    for i in range(3):
        for j in range(3):
            acc = a[3 * i + 0] * b[0 + j]
            acc = acc + a[3 * i + 1] * b[3 + j]
            acc = acc + a[3 * i + 2] * b[6 + j]
            out.append(acc)
    return out


def _final_kernel(mc_ref, p_ref, xn_ref, q_ref, w3_ref, w4_ref, w5_ref,
                  out_ref):
    msgp = _planes_from_packed(mc_ref[...])
    yp = _planes_from_packed(p_ref[...])
    q = 1.0 + 0.1 * q_ref[...]
    a2 = _matmul3(msgp, yp)
    b2 = _matmul3(yp, msgp)
    t = [q * (a2[k] + b2[k]) for k in range(9)]
    norm = t[0] * t[0]
    for k in range(1, 9):
        norm = norm + t[k] * t[k]
    inv = 1.0 / (norm + 1.0)
    dm = (t[0] + t[4] + t[8]) * (1.0 / 3.0) * inv
    a01 = 0.5 * (t[1] - t[3]) * inv
    a02 = 0.5 * (t[2] - t[6]) * inv
    a12 = 0.5 * (t[5] - t[7]) * inv
    s00 = t[0] * inv - dm
    s11 = t[4] * inv - dm
    s01 = 0.5 * (t[1] + t[3]) * inv
    s02 = 0.5 * (t[2] + t[6]) * inv
    s12 = 0.5 * (t[5] + t[7]) * inv
    w3 = w3_ref[...]
    w4 = w4_ref[...]
    w5 = w5_ref[...]

    def mm(v, w):
        return jnp.dot(v, w, preferred_element_type=jnp.float32)

    i2 = mm(dm, w3)
    a01, a02, a12 = mm(a01, w4), mm(a02, w4), mm(a12, w4)
    s00, s01, s02, s11, s12 = (mm(s00, w5), mm(s01, w5), mm(s02, w5),
                               mm(s11, w5), mm(s12, w5))
    s22 = -(s00 + s11)
    dx = [i2 + s00, a01 + s01, a02 + s02,
          -a01 + s01, i2 + s11, a12 + s12,
          -a02 + s02, -a12 + s12, i2 + s22]
    dxdx = _matmul3(dx, dx)
    xn = xn_ref[...]
    for k in range(9):
        out_ref[k, :, :] = xn[k] + dx[k] + q * dxdx[k]


def kernel(X, pair_indices, d_ij, radial_feature_vector, atomic_charges,
           W1, b1, W2, b2, W3, b3, Wt):
    n, f = X.shape[0], X.shape[1]
    e = pair_indices.shape[1]
    assert f == _F
    bn = _largest_divisor_leq(n, 1024)
    eb = _largest_divisor_leq(e, 1024)

    xc = X.transpose(2, 3, 0, 1).reshape(9, n, f)
    perm = np.arange(3 * f).reshape(f, 3).T.reshape(-1)
    w3p = W3[:, perm]
    b3p = b3[perm].reshape(1, 3 * f)
    b1r = b1.reshape(1, -1)
    b2r = b2.reshape(1, -1)
    sc3 = pair_indices[0].reshape(e // eb, 1, eb)
    gt3 = pair_indices[1].reshape(e // eb, 1, eb)
    qv = atomic_charges.reshape(n, 1)

    const2 = lambda i: (0, 0)
    const3 = lambda i: (0, 0, 0)

    p, xn = pl.pallas_call(
        _node_prep_kernel,
        grid=(n // bn,),
        in_specs=[
            pl.BlockSpec((9, bn, f), lambda i: (0, i, 0)),
            pl.BlockSpec((f, f), const2),
            pl.BlockSpec((f, f), const2),
            pl.BlockSpec((f, f), const2),
        ],
        out_specs=[
            pl.BlockSpec((bn, _D), lambda i: (i, 0)),
            pl.BlockSpec((9, bn, f), lambda i: (0, i, 0)),
        ],
        out_shape=[
            jax.ShapeDtypeStruct((n, _D), jnp.float32),
            jax.ShapeDtypeStruct((9, n, f), jnp.float32),
        ],
    )(xc, Wt[0], Wt[1], Wt[2])

    mc = pl.pallas_call(
        _msg_kernel,
        grid=(e // eb,),
        in_specs=[
            pl.BlockSpec((eb, radial_feature_vector.shape[1]),
                         lambda i: (i, 0)),
            pl.BlockSpec((eb, 1), lambda i: (i, 0)),
            pl.BlockSpec((1, 1, eb), lambda i: (i, 0, 0),
                         memory_space=pltpu.SMEM),
            pl.BlockSpec((1, 1, eb), lambda i: (i, 0, 0),
                         memory_space=pltpu.SMEM),
            pl.BlockSpec((n, _D), const2),
            pl.BlockSpec(W1.shape, const2),
            pl.BlockSpec((1, W1.shape[1]), const2),
            pl.BlockSpec(W2.shape, const2),
            pl.BlockSpec((1, W2.shape[1]), const2),
            pl.BlockSpec(w3p.shape, const2),
            pl.BlockSpec((1, w3p.shape[1]), const2),
        ],
        out_specs=pl.BlockSpec((n, _D), const2),
        out_shape=jax.ShapeDtypeStruct((n, _D), jnp.float32),
        scratch_shapes=[pltpu.VMEM((eb, _D), jnp.float32)],
    )(radial_feature_vector, d_ij, sc3, gt3, p,
      W1, b1r, W2, b2r, w3p, b3p)

    xout9 = pl.pallas_call(
        _final_kernel,
        grid=(n // bn,),
        in_specs=[
            pl.BlockSpec((bn, _D), lambda i: (i, 0)),
            pl.BlockSpec((bn, _D), lambda i: (i, 0)),
            pl.BlockSpec((9, bn, f), lambda i: (0, i, 0)),
            pl.BlockSpec((bn, 1), lambda i: (i, 0)),
            pl.BlockSpec((f, f), const2),
            pl.BlockSpec((f, f), const2),
            pl.BlockSpec((f, f), const2),
        ],
        out_specs=pl.BlockSpec((9, bn, f), lambda i: (0, i, 0)),
        out_shape=jax.ShapeDtypeStruct((9, n, f), jnp.float32),
    )(mc, p, xn, qv, Wt[3], Wt[4], Wt[5])

    return xout9.transpose(1, 2, 0).reshape(n, f, 3, 3)


# speedup vs baseline: 40.3898x; 1.3889x over previous
import numpy as np
import jax
import jax.numpy as jnp
from jax.experimental import pallas as pl
from jax.experimental.pallas import tpu as pltpu

_F = 64
_D = 9 * _F
_CUT = 5.0


def _largest_divisor_leq(n, cap):
    for d in range(min(cap, n), 0, -1):
        if n % d == 0:
            return d
    return 1


def _node_prep_kernel(xc_ref, w0_ref, w1_ref, w2_ref, p_ref, xn_ref):
    x = xc_ref[...]
    norm = jnp.sum(x * x, axis=0)
    inv = 1.0 / (norm + 1.0)
    xn = x * inv[None]
    xn_ref[...] = xn
    dm = (xn[0] + xn[4] + xn[8]) * (1.0 / 3.0)
    a01 = 0.5 * (xn[1] - xn[3])
    a02 = 0.5 * (xn[2] - xn[6])
    a12 = 0.5 * (xn[5] - xn[7])
    s00 = xn[0] - dm
    s11 = xn[4] - dm
    s01 = 0.5 * (xn[1] + xn[3])
    s02 = 0.5 * (xn[2] + xn[6])
    s12 = 0.5 * (xn[5] + xn[7])
    w0 = w0_ref[...]
    w1 = w1_ref[...]
    w2 = w2_ref[...]

    def mm(v, w):
        return jnp.dot(v, w, preferred_element_type=jnp.float32)

    p_ref[:, 0 * _F:1 * _F] = mm(dm, w0)
    p_ref[:, 1 * _F:2 * _F] = mm(a01, w1)
    p_ref[:, 2 * _F:3 * _F] = mm(a02, w1)
    p_ref[:, 3 * _F:4 * _F] = mm(a12, w1)
    p_ref[:, 4 * _F:5 * _F] = mm(s00, w2)
    p_ref[:, 5 * _F:6 * _F] = mm(s01, w2)
    p_ref[:, 6 * _F:7 * _F] = mm(s02, w2)
    p_ref[:, 7 * _F:8 * _F] = mm(s11, w2)
    p_ref[:, 8 * _F:9 * _F] = mm(s12, w2)


def _msg_kernel(rbf_ref, d_ref, sc_ref, gt_ref, p_ref,
                w1_ref, b1_ref, w2_ref, b2_ref, w3_ref, b3_ref, mc_ref,
                scale_ref):
    @pl.when(pl.program_id(0) == 0)
    def _init():
        mc_ref[...] = jnp.zeros_like(mc_ref)

    h = rbf_ref[...]
    h = jax.nn.silu(jnp.dot(h, w1_ref[...], preferred_element_type=jnp.float32)
                    + b1_ref[...])
    h = jax.nn.silu(jnp.dot(h, w2_ref[...], preferred_element_type=jnp.float32)
                    + b2_ref[...])
    h = jax.nn.silu(jnp.dot(h, w3_ref[...], preferred_element_type=jnp.float32)
                    + b3_ref[...])
    d = d_ref[...]
    c = 0.5 * (jnp.cos(d * (np.pi / _CUT)) + 1.0)
    c = c * (d < _CUT).astype(jnp.float32)
    r = h * c
    r0 = r[:, 0 * _F:1 * _F]
    r1 = r[:, 1 * _F:2 * _F]
    r2 = r[:, 2 * _F:3 * _F]
    scale_ref[...] = jnp.concatenate([r0, r1, r1, r1, r2, r2, r2, r2, r2],
                                     axis=1)
    eb = r.shape[0]

    def body(i, carry):
        g = gt_ref[0, 0, i]
        s = sc_ref[0, 0, i]
        row = p_ref[pl.ds(g, 1), :]
        m = row * scale_ref[pl.ds(i, 1), :]
        mc_ref[pl.ds(s, 1), :] = mc_ref[pl.ds(s, 1), :] + m
        return carry

    jax.lax.fori_loop(0, eb, body, 0, unroll=8)


def _planes_from_packed(p):
    i = p[:, 0 * _F:1 * _F]
    a01 = p[:, 1 * _F:2 * _F]
    a02 = p[:, 2 * _F:3 * _F]
    a12 = p[:, 3 * _F:4 * _F]
    s00 = p[:, 4 * _F:5 * _F]
    s01 = p[:, 5 * _F:6 * _F]
    s02 = p[:, 6 * _F:7 * _F]
    s11 = p[:, 7 * _F:8 * _F]
    s12 = p[:, 8 * _F:9 * _F]
    s22 = -(s00 + s11)
    return [i + s00, a01 + s01, a02 + s02,
            -a01 + s01, i + s11, a12 + s12,
            -a02 + s02, -a12 + s12, i + s22]


def _matmul3(a, b):
    out = []
    for i in range(3):
        for j in range(3):
            acc = a[3 * i + 0] * b[0 + j]
            acc = acc + a[3 * i + 1] * b[3 + j]
            acc = acc + a[3 * i + 2] * b[6 + j]
            out.append(acc)
    return out


def _final_kernel(mc_ref, p_ref, xn_ref, q_ref, w3_ref, w4_ref, w5_ref,
                  out_ref):
    msgp = _planes_from_packed(mc_ref[...])
    yp = _planes_from_packed(p_ref[...])
    q = 1.0 + 0.1 * q_ref[...]
    a2 = _matmul3(msgp, yp)
    b2 = _matmul3(yp, msgp)
    t = [q * (a2[k] + b2[k]) for k in range(9)]
    norm = t[0] * t[0]
    for k in range(1, 9):
        norm = norm + t[k] * t[k]
    inv = 1.0 / (norm + 1.0)
    dm = (t[0] + t[4] + t[8]) * (1.0 / 3.0) * inv
    a01 = 0.5 * (t[1] - t[3]) * inv
    a02 = 0.5 * (t[2] - t[6]) * inv
    a12 = 0.5 * (t[5] - t[7]) * inv
    s00 = t[0] * inv - dm
    s11 = t[4] * inv - dm
    s01 = 0.5 * (t[1] + t[3]) * inv
    s02 = 0.5 * (t[2] + t[6]) * inv
    s12 = 0.5 * (t[5] + t[7]) * inv
    w3 = w3_ref[...]
    w4 = w4_ref[...]
    w5 = w5_ref[...]

    def mm(v, w):
        return jnp.dot(v, w, preferred_element_type=jnp.float32)

    i2 = mm(dm, w3)
    a01, a02, a12 = mm(a01, w4), mm(a02, w4), mm(a12, w4)
    s00, s01, s02, s11, s12 = (mm(s00, w5), mm(s01, w5), mm(s02, w5),
                               mm(s11, w5), mm(s12, w5))
    s22 = -(s00 + s11)
    dx = [i2 + s00, a01 + s01, a02 + s02,
          -a01 + s01, i2 + s11, a12 + s12,
          -a02 + s02, -a12 + s12, i2 + s22]
    dxdx = _matmul3(dx, dx)
    xn = xn_ref[...]
    for k in range(9):
        out_ref[k, :, :] = xn[k] + dx[k] + q * dxdx[k]


def kernel(X, pair_indices, d_ij, radial_feature_vector, atomic_charges,
           W1, b1, W2, b2, W3, b3, Wt):
    n, f = X.shape[0], X.shape[1]
    e = pair_indices.shape[1]
    assert f == _F
    bn = _largest_divisor_leq(n, 1024)
    eb = _largest_divisor_leq(e, 1024)

    xc = X.transpose(2, 3, 0, 1).reshape(9, n, f)
    perm = np.arange(3 * f).reshape(f, 3).T.reshape(-1)
    w3p = W3[:, perm]
    b3p = b3[perm].reshape(1, 3 * f)
    b1r = b1.reshape(1, -1)
    b2r = b2.reshape(1, -1)
    sc3 = pair_indices[0].reshape(e // eb, 1, eb)
    gt3 = pair_indices[1].reshape(e // eb, 1, eb)
    qv = atomic_charges.reshape(n, 1)

    const2 = lambda i: (0, 0)
    const3 = lambda i: (0, 0, 0)

    p, xn = pl.pallas_call(
        _node_prep_kernel,
        grid=(n // bn,),
        in_specs=[
            pl.BlockSpec((9, bn, f), lambda i: (0, i, 0)),
            pl.BlockSpec((f, f), const2),
            pl.BlockSpec((f, f), const2),
            pl.BlockSpec((f, f), const2),
        ],
        out_specs=[
            pl.BlockSpec((bn, _D), lambda i: (i, 0)),
            pl.BlockSpec((9, bn, f), lambda i: (0, i, 0)),
        ],
        out_shape=[
            jax.ShapeDtypeStruct((n, _D), jnp.float32),
            jax.ShapeDtypeStruct((9, n, f), jnp.float32),
        ],
    )(xc, Wt[0], Wt[1], Wt[2])

    mc = pl.pallas_call(
        _msg_kernel,
        grid=(e // eb,),
        in_specs=[
            pl.BlockSpec((eb, radial_feature_vector.shape[1]),
                         lambda i: (i, 0)),
            pl.BlockSpec((eb, 1), lambda i: (i, 0)),
            pl.BlockSpec((1, 1, eb), lambda i: (i, 0, 0),
                         memory_space=pltpu.SMEM),
            pl.BlockSpec((1, 1, eb), lambda i: (i, 0, 0),
                         memory_space=pltpu.SMEM),
            pl.BlockSpec((n, _D), const2),
            pl.BlockSpec(W1.shape, const2),
            pl.BlockSpec((1, W1.shape[1]), const2),
            pl.BlockSpec(W2.shape, const2),
            pl.BlockSpec((1, W2.shape[1]), const2),
            pl.BlockSpec(w3p.shape, const2),
            pl.BlockSpec((1, w3p.shape[1]), const2),
        ],
        out_specs=pl.BlockSpec((n, _D), const2),
        out_shape=jax.ShapeDtypeStruct((n, _D), jnp.float32),
        scratch_shapes=[pltpu.VMEM((eb, _D), jnp.float32)],
    )(radial_feature_vector, d_ij, sc3, gt3, p,
      W1, b1r, W2, b2r, w3p, b3p)

    xout9 = pl.pallas_call(
        _final_kernel,
        grid=(n // bn,),
        in_specs=[
            pl.BlockSpec((bn, _D), lambda i: (i, 0)),
            pl.BlockSpec((bn, _D), lambda i: (i, 0)),
            pl.BlockSpec((9, bn, f), lambda i: (0, i, 0)),
            pl.BlockSpec((bn, 1), lambda i: (i, 0)),
            pl.BlockSpec((f, f), const2),
            pl.BlockSpec((f, f), const2),
            pl.BlockSpec((f, f), const2),
        ],
        out_specs=pl.BlockSpec((9, bn, f), lambda i: (0, i, 0)),
        out_shape=jax.ShapeDtypeStruct((9, n, f), jnp.float32),
    )(mc, p, xn, qv, Wt[3], Wt[4], Wt[5])

    return xout9.transpose(1, 2, 0).reshape(n, f, 3, 3)
